# Initial kernel scaffold; baseline (speedup 1.0000x reference)
import jax
import jax.numpy as jnp
from jax.experimental import pallas as pl


def kernel(input_ids, token_type_ids, word_emb, pos_emb, type_emb, ln_w, ln_b):
    raise NotImplementedError("write your pallas kernel here")


# speedup vs baseline: 2.3060x; 2.3060x over previous
import functools

import jax
import jax.numpy as jnp
from jax import lax
from jax.experimental import pallas as pl
from jax.experimental.pallas import tpu as pltpu
from jax.experimental.pallas import tpu_sc as plsc

_EPS = 1e-12
_H = 128
_NV = _H // 16


def _rsqrt(v):
    i = lax.bitcast_convert_type(v, jnp.int32)
    i = jnp.int32(0x5F3759DF) - lax.shift_right_logical(i, 1)
    y = lax.bitcast_convert_type(i, jnp.float32)
    for _ in range(3):
        y = y * (1.5 - 0.5 * v * y * y)
    return y


def _make_sc_kernel(B, L, H):
    info = plsc.get_sparse_core_info()
    NC, NS = info.num_cores, info.num_subcores
    NW = NC * NS
    assert B % NW == 0
    rows_per_w = B // NW

    mesh = plsc.VectorSubcoreMesh(core_axis_name="c", subcore_axis_name="s")

    @functools.partial(
        pl.kernel,
        mesh=mesh,
        compiler_params=pltpu.CompilerParams(needs_layout_passes=False),
        out_type=jax.ShapeDtypeStruct((B, L, H), jnp.float32),
        scratch_types=[
            pltpu.VMEM((L,), jnp.int32),
            pltpu.VMEM((L,), jnp.int32),
            pltpu.VMEM((L, H), jnp.float32),
            pltpu.VMEM((L, H), jnp.float32),
            pltpu.VMEM((L, H), jnp.float32),
            pltpu.VMEM((2, H), jnp.float32),
            pltpu.VMEM((H,), jnp.float32),
            pltpu.VMEM((H,), jnp.float32),
            pltpu.SemaphoreType.DMA,
        ],
    )
    def sc_kernel(ids_hbm, tt_hbm, word_hbm, pos_hbm, type_hbm, lnw_hbm,
                  lnb_hbm, out_hbm, idx_v, tt_v, rows_v, out_v, pos_v,
                  type_v, lnw_v, lnb_v, sem):
        wid = lax.axis_index("s") * NC + lax.axis_index("c")

        pltpu.sync_copy(pos_hbm.at[pl.ds(0, L)], pos_v)
        pltpu.sync_copy(type_hbm, type_v)
        pltpu.sync_copy(lnw_hbm, lnw_v)
        pltpu.sync_copy(lnb_hbm, lnb_v)

        t0 = [type_v[0, pl.ds(16 * d, 16)] for d in range(_NV)]
        t1 = [type_v[1, pl.ds(16 * d, 16)] for d in range(_NV)]
        dt = [t1[d] - t0[d] for d in range(_NV)]

        def add_t0(p, _):
            for d in range(_NV):
                pos_v[p, pl.ds(16 * d, 16)] = pos_v[p, pl.ds(16 * d, 16)] + t0[d]
            return 0
        lax.fori_loop(0, L, add_t0, 0)

        lnw = [lnw_v[pl.ds(16 * d, 16)] for d in range(_NV)]
        lnb = [lnb_v[pl.ds(16 * d, 16)] for d in range(_NV)]
        inv_h = jnp.float32(1.0 / H)

        def row_body(r, _):
            row = wid * rows_per_w + r
            pltpu.sync_copy(ids_hbm.at[row], idx_v)
            pltpu.sync_copy(tt_hbm.at[row], tt_v)
            pltpu.async_copy(word_hbm.at[idx_v], rows_v, sem).wait()

            def tok(t, _):
                ttf = plsc.load_gather(
                    tt_v, [jnp.full((16,), t, jnp.int32)]).astype(jnp.float32)
                x = [rows_v[t, pl.ds(16 * d, 16)]
                     + pos_v[t, pl.ds(16 * d, 16)]
                     + ttf * dt[d]
                     for d in range(_NV)]
                s = x[0]
                for d in range(1, _NV):
                    s = s + x[d]
                u = jnp.sum(s) * inv_h
                xc = [x[d] - u for d in range(_NV)]
                sq = xc[0] * xc[0]
                for d in range(1, _NV):
                    sq = sq + xc[d] * xc[d]
                var = jnp.sum(sq) * inv_h
                inv = _rsqrt(var + _EPS)
                for d in range(_NV):
                    out_v[t, pl.ds(16 * d, 16)] = xc[d] * inv * lnw[d] + lnb[d]
                return 0

            lax.fori_loop(0, L, tok, 0)
            pltpu.sync_copy(out_v, out_hbm.at[row])
            return 0

        lax.fori_loop(0, rows_per_w, row_body, 0)

    return sc_kernel


@jax.jit
def kernel(input_ids, token_type_ids, word_emb, pos_emb, type_emb, ln_w, ln_b):
    B, L = input_ids.shape
    H = word_emb.shape[1]
    ids = input_ids.astype(jnp.int32)
    tts = token_type_ids.astype(jnp.int32)
    fn = _make_sc_kernel(B, L, H)
    return fn(ids, tts, word_emb, pos_emb, type_emb, ln_w, ln_b)


# speedup vs baseline: 6.9146x; 2.9985x over previous
import functools

import jax
import jax.numpy as jnp
from jax import lax
from jax.experimental import pallas as pl
from jax.experimental.pallas import tpu as pltpu
from jax.experimental.pallas import tpu_sc as plsc

_EPS = 1e-12
_H = 128
_NV = _H // 16


def _rsqrt(v):
    i = lax.bitcast_convert_type(v, jnp.int32)
    i = jnp.int32(0x5F3759DF) - lax.shift_right_logical(i, 1)
    y = lax.bitcast_convert_type(i, jnp.float32)
    for _ in range(3):
        y = y * (1.5 - 0.5 * v * y * y)
    return y


def _make_sc_kernel(B, L, H):
    info = plsc.get_sparse_core_info()
    NC, NS = info.num_cores, info.num_subcores
    NW = NC * NS
    assert B % NW == 0
    rows_per_w = B // NW

    mesh = plsc.VectorSubcoreMesh(core_axis_name="c", subcore_axis_name="s")

    @functools.partial(
        pl.kernel,
        mesh=mesh,
        compiler_params=pltpu.CompilerParams(needs_layout_passes=False),
        out_type=jax.ShapeDtypeStruct((B, L, H), jnp.float32),
        scratch_types=[
            pltpu.VMEM((L,), jnp.int32),
            pltpu.VMEM((L,), jnp.int32),
            pltpu.VMEM((2, L), jnp.int32),
            pltpu.VMEM((L, H), jnp.float32),
            pltpu.VMEM((L, H), jnp.float32),
            pltpu.VMEM((L, H), jnp.float32),
            pltpu.VMEM((L, H), jnp.float32),
            pltpu.VMEM((2, H), jnp.float32),
            pltpu.VMEM((H,), jnp.float32),
            pltpu.VMEM((H,), jnp.float32),
            pltpu.SemaphoreType.DMA,
            pltpu.SemaphoreType.DMA,
        ],
    )
    def sc_kernel(ids_hbm, tt_hbm, word_hbm, pos_hbm, type_hbm, lnw_hbm,
                  lnb_hbm, out_hbm, idx0_v, idx1_v, tt_v, rows0_v, rows1_v,
                  out_v, pos_v, type_v, lnw_v, lnb_v, sem0, sem1):
        wid = lax.axis_index("s") * NC + lax.axis_index("c")
        base = wid * rows_per_w
        sems = (sem0, sem1)
        idxs = (idx0_v, idx1_v)
        rows = (rows0_v, rows1_v)

        pltpu.sync_copy(pos_hbm.at[pl.ds(0, L)], pos_v)
        pltpu.sync_copy(type_hbm, type_v)
        pltpu.sync_copy(lnw_hbm, lnw_v)
        pltpu.sync_copy(lnb_hbm, lnb_v)

        t0 = [type_v[0, pl.ds(16 * d, 16)] for d in range(_NV)]
        t1 = [type_v[1, pl.ds(16 * d, 16)] for d in range(_NV)]
        dt = [t1[d] - t0[d] for d in range(_NV)]

        @plsc.parallel_loop(0, L)
        def _(p):
            for d in range(_NV):
                pos_v[p, pl.ds(16 * d, 16)] = pos_v[p, pl.ds(16 * d, 16)] + t0[d]

        lnw = [lnw_v[pl.ds(16 * d, 16)] for d in range(_NV)]
        lnb = [lnb_v[pl.ds(16 * d, 16)] for d in range(_NV)]
        inv_h = jnp.float32(1.0 / H)

        def start_gather(r, k):
            pltpu.sync_copy(ids_hbm.at[base + r], idxs[k])
            pltpu.sync_copy(tt_hbm.at[base + r], tt_v.at[k])
            pltpu.async_copy(word_hbm.at[idxs[k]], rows[k], sems[k])

        def compute_row(r, k):
            pltpu.make_async_copy(
                word_hbm.at[idxs[k]], rows[k], sems[k]).wait()

            @plsc.parallel_loop(0, L, unroll=4)
            def _(t):
                ttf = plsc.load_gather(
                    tt_v, [jnp.full((16,), k, jnp.int32),
                           jnp.full((16,), t, jnp.int32)]
                ).astype(jnp.float32)
                x = [rows[k][t, pl.ds(16 * d, 16)]
                     + pos_v[t, pl.ds(16 * d, 16)]
                     + ttf * dt[d]
                     for d in range(_NV)]
                s = x[0] + x[1]
                sq = x[0] * x[0] + x[1] * x[1]
                for d in range(2, _NV):
                    s = s + x[d]
                    sq = sq + x[d] * x[d]
                u = jnp.sum(s) * inv_h
                msq = jnp.sum(sq) * inv_h
                var = msq - u * u
                inv = _rsqrt(var + _EPS)
                c = u * inv
                for d in range(_NV):
                    out_v[t, pl.ds(16 * d, 16)] = (
                        (x[d] * inv - c) * lnw[d] + lnb[d])

            pltpu.sync_copy(out_v, out_hbm.at[base + r])

        start_gather(0, 0)

        def pair_body(p, _):
            r = 2 * p
            for k in range(2):

                @pl.when(r + k + 1 < rows_per_w)
                def _():
                    start_gather(r + k + 1, 1 - k)

                compute_row(r + k, k)
            return 0

        lax.fori_loop(0, rows_per_w // 2, pair_body, 0)

    return sc_kernel


@jax.jit
def kernel(input_ids, token_type_ids, word_emb, pos_emb, type_emb, ln_w, ln_b):
    B, L = input_ids.shape
    H = word_emb.shape[1]
    ids = input_ids.astype(jnp.int32)
    tts = token_type_ids.astype(jnp.int32)
    fn = _make_sc_kernel(B, L, H)
    return fn(ids, tts, word_emb, pos_emb, type_emb, ln_w, ln_b)
